# baseline (speedup 1.0000x reference)
import dataclasses
import functools

import jax
import jax.numpy as jnp
from jax import lax
from jax.experimental import pallas as pl
from jax.experimental.pallas import tpu as pltpu
from jax.experimental.pallas import tpu_sc as plsc

NUM_CORES = 2
NUM_SUBCORES = 16
LANES = 16
NUM_TILES = NUM_CORES * NUM_SUBCORES

BATCH = 16384
DIM = 64
ROWS_PER_TILE = BATCH // NUM_TILES
CHUNK = 128
NUM_CHUNKS = ROWS_PER_TILE // CHUNK
GROUPS = ROWS_PER_TILE // LANES


def _mf_kernel(users_hbm, items_hbm, utab_hbm, itab_hbm, out_hbm,
               uidx_v, iidx_v, urows_v, irows_v, out_v, sem):
    wid = lax.axis_index("s") * NUM_CORES + lax.axis_index("c")

    pltpu.sync_copy(users_hbm.at[wid], uidx_v)
    pltpu.sync_copy(items_hbm.at[wid], iidx_v)

    copies = []
    for j in range(NUM_CHUNKS):
        dst = urows_v.at[pl.ds(j * CHUNK, CHUNK)]
        copies.append(pltpu.async_copy(utab_hbm.at[uidx_v.at[j]], dst, sem))
        dst = irows_v.at[pl.ds(j * CHUNK, CHUNK)]
        copies.append(pltpu.async_copy(itab_hbm.at[iidx_v.at[j]], dst, sem))
    for c in copies:
        c.wait()

    row_iota = lax.iota(jnp.int32, LANES)

    @pl.loop(0, GROUPS)
    def _(g):
        rows = g * LANES + row_iota
        accs = [jnp.zeros((LANES,), jnp.float32) for _ in range(4)]
        for d in range(DIM):
            col = jnp.full((LANES,), d, jnp.int32)
            u = plsc.load_gather(urows_v, [rows, col])
            v = plsc.load_gather(irows_v, [rows, col])
            accs[d % 4] = accs[d % 4] + u * v
        s = (accs[0] + accs[1]) + (accs[2] + accs[3])
        out_v[pl.ds(g * LANES, LANES)] = 1.0 / (1.0 + jnp.exp(-s))

    pltpu.sync_copy(out_v, out_hbm.at[wid])


@jax.jit
def kernel(users, items, user_table, item_table):
    users = users.reshape(NUM_TILES, NUM_CHUNKS, CHUNK)
    items = items.reshape(NUM_TILES, NUM_CHUNKS, CHUNK)
    mesh = plsc.VectorSubcoreMesh(core_axis_name="c", subcore_axis_name="s")
    cp = pltpu.CompilerParams(
        needs_layout_passes=False, use_tc_tiling_on_sc=False
    )
    mf = pl.kernel(
        _mf_kernel,
        out_type=jax.ShapeDtypeStruct((NUM_TILES, ROWS_PER_TILE), jnp.float32),
        mesh=mesh,
        scratch_types=[
            pltpu.VMEM((NUM_CHUNKS, CHUNK), jnp.int32),
            pltpu.VMEM((NUM_CHUNKS, CHUNK), jnp.int32),
            pltpu.VMEM((ROWS_PER_TILE, DIM), jnp.float32),
            pltpu.VMEM((ROWS_PER_TILE, DIM), jnp.float32),
            pltpu.VMEM((ROWS_PER_TILE,), jnp.float32),
            pltpu.SemaphoreType.DMA,
        ],
        compiler_params=cp,
    )
    out = mf(users, items, user_table, item_table)
    return out.reshape(BATCH)
